# baseline (speedup 1.0000x reference)
import functools

import jax
import jax.numpy as jnp
from jax import lax
from jax.experimental import pallas as pl
from jax.experimental.pallas import tpu as pltpu
from jax.experimental.pallas import tpu_sc as plsc

LANES = 16


def _build_encoder(batch, seq, vocab, dim):
    info = plsc.get_sparse_core_info()
    nc, ns = info.num_cores, info.num_subcores
    nw = nc * ns
    dblk = dim // 8
    bblk = batch // 128
    ut = 512
    uj = ut // 128
    upos = batch // ut
    n_units = seq * upos
    upw = n_units // nw
    gsz = 128
    n_gather = ut // gsz

    mesh = plsc.VectorSubcoreMesh(core_axis_name="c", subcore_axis_name="s")

    @functools.partial(
        pl.kernel,
        mesh=mesh,
        compiler_params=pltpu.CompilerParams(
            use_tc_tiling_on_sc=False, needs_layout_passes=False),
        out_type=jax.ShapeDtypeStruct((seq, dblk * bblk * 8, 128), jnp.float32),
        scratch_types=[
            pltpu.VMEM((ut,), jnp.int32),
            pltpu.VMEM((ut,), jnp.int32),
            pltpu.VMEM((ut, dim), jnp.float32),
            pltpu.VMEM((ut, dim), jnp.float32),
            pltpu.VMEM((dblk * uj * 8, 129), jnp.float32),
            pltpu.VMEM((dblk * uj * 8, 129), jnp.float32),
            pltpu.VMEM((seq, dim), jnp.float32),
            pltpu.SemaphoreType.DMA,
            pltpu.SemaphoreType.DMA,
            pltpu.SemaphoreType.DMA,
            pltpu.SemaphoreType.DMA,
            pltpu.SemaphoreType.DMA,
            pltpu.SemaphoreType.DMA,
        ],
    )
    def enc(ids_hbm, table_hbm, pos_hbm, out_hbm,
            idx_v0, idx_v1, rows_v0, rows_v1, ob0, ob1, pos_v,
            sg0, sg1, si0, si1, so0, so1):
        idx_v = (idx_v0, idx_v1)
        rows_v = (rows_v0, rows_v1)
        obuf = (ob0, ob1)
        sg = (sg0, sg1)
        si = (si0, si1)
        so = (so0, so1)
        wid = lax.axis_index("s") * nc + lax.axis_index("c")
        u0 = wid * upw
        pltpu.sync_copy(pos_hbm, pos_v)

        def unit_su(n):
            u = u0 + n
            return u // upos, (u % upos) * uj

        def ids_src(n):
            s, j0 = unit_su(n)
            return ids_hbm.at[s, pl.ds(j0 * 128, ut)]

        def out_pairs(p, n):
            s, j0 = unit_su(n)
            return [
                (obuf[p].at[pl.ds(i * 8 * uj, 8 * uj), pl.ds(0, 128)],
                 out_hbm.at[s, pl.ds(i * bblk * 8 + j0 * 8, 8 * uj)])
                for i in range(dblk)
            ]

        def fire_out(p, n):
            for src, dst in out_pairs(p, n):
                pltpu.async_copy(src, dst, so[p])

        def wait_out(p, n):
            for src, dst in out_pairs(p, n):
                pltpu.make_async_copy(src, dst, so[p]).wait()

        def fire_gathers(p):
            for k in range(n_gather):
                pltpu.async_copy(
                    table_hbm.at[idx_v[p].at[pl.ds(k * gsz, gsz)]],
                    rows_v[p].at[pl.ds(k * gsz, gsz)],
                    sg[p],
                )

        def wait_gathers(p):
            for k in range(n_gather):
                pltpu.make_async_copy(
                    table_hbm.at[idx_v[p].at[pl.ds(k * gsz, gsz)]],
                    rows_v[p].at[pl.ds(k * gsz, gsz)],
                    sg[p],
                ).wait()

        iota = lax.iota(jnp.int32, LANES)
        row_pat = (iota // 8) * (8 * uj) + (iota % 8)

        def transpose_add(p, n):
            s, _ = unit_su(n)
            pv0 = pos_v[s, pl.ds(0, LANES)]
            pv1 = pos_v[s, pl.ds(LANES, LANES)]

            @plsc.parallel_loop(0, ut, unroll=8)
            def body(t):
                row0 = row_pat + jnp.full((LANES,), (t // 128) * 8, jnp.int32)
                row1 = row0 + 2 * (8 * uj)
                col = jnp.full((LANES,), t % 128, jnp.int32)
                v0 = rows_v[p][t, pl.ds(0, LANES)] + pv0
                v1 = rows_v[p][t, pl.ds(LANES, LANES)] + pv1
                plsc.store_scatter(obuf[p], [row0, col], v0)
                plsc.store_scatter(obuf[p], [row1, col], v1)

        pltpu.sync_copy(ids_src(0), idx_v[0])
        fire_gathers(0)
        pltpu.async_copy(ids_src(1), idx_v[1], si[1])

        def step(n, p):
            wait_gathers(p)

            @pl.when(n + 1 < upw)
            def _():
                pltpu.make_async_copy(ids_src(n + 1), idx_v[1 - p], si[1 - p]).wait()
                fire_gathers(1 - p)

            @pl.when(n + 2 < upw)
            def _():
                pltpu.async_copy(ids_src(n + 2), idx_v[p], si[p])

            @pl.when(n >= 2)
            def _():
                wait_out(p, n - 2)

            transpose_add(p, n)
            fire_out(p, n)

        def pair(m, carry):
            step(2 * m, 0)
            step(2 * m + 1, 1)
            return carry

        lax.fori_loop(0, upw // 2, pair, 0)
        wait_out(0, upw - 2)
        wait_out(1, upw - 1)

    return enc


def kernel(token_ids, token_table, pos_table):
    batch, seq = token_ids.shape
    vocab, dim = token_table.shape
    enc = _build_encoder(batch, seq, vocab, dim)
    ids_t = jnp.transpose(token_ids)
    out3 = enc(ids_t, token_table, pos_table)
    out5 = out3.reshape(seq, dim // 8, batch // 128, 8, 128)
    out = jnp.transpose(out5, (2, 4, 0, 1, 3)).reshape(batch, seq, dim)
    return out


# speedup vs baseline: 1.0035x; 1.0035x over previous
import functools

import jax
import jax.numpy as jnp
from jax import lax
from jax.experimental import pallas as pl
from jax.experimental.pallas import tpu as pltpu
from jax.experimental.pallas import tpu_sc as plsc

LANES = 16


def _build_encoder(batch, seq, vocab, dim):
    info = plsc.get_sparse_core_info()
    nc, ns = info.num_cores, info.num_subcores
    nw = nc * ns
    dblk = dim // 8
    bblk = batch // 128
    ut = 512
    uj = ut // 128
    upos = batch // ut
    n_units = seq * upos
    upw = n_units // nw
    gsz = 128
    n_gather = ut // gsz

    mesh = plsc.VectorSubcoreMesh(core_axis_name="c", subcore_axis_name="s")

    @functools.partial(
        pl.kernel,
        mesh=mesh,
        compiler_params=pltpu.CompilerParams(
            use_tc_tiling_on_sc=False, needs_layout_passes=False),
        out_type=jax.ShapeDtypeStruct((seq, dblk * bblk * 8, 128), jnp.float32),
        scratch_types=[
            pltpu.VMEM((ut,), jnp.int32),
            pltpu.VMEM((ut,), jnp.int32),
            pltpu.VMEM((ut, dim), jnp.float32),
            pltpu.VMEM((ut, dim), jnp.float32),
            pltpu.VMEM((dblk * uj * 8, 129), jnp.float32),
            pltpu.VMEM((dblk * uj * 8, 129), jnp.float32),
            pltpu.VMEM((seq, dim), jnp.float32),
            pltpu.SemaphoreType.DMA,
            pltpu.SemaphoreType.DMA,
            pltpu.SemaphoreType.DMA,
            pltpu.SemaphoreType.DMA,
            pltpu.SemaphoreType.DMA,
            pltpu.SemaphoreType.DMA,
        ],
    )
    def enc(ids_hbm, table_hbm, pos_hbm, out_hbm,
            idx_v0, idx_v1, rows_v0, rows_v1, ob0, ob1, pos_v,
            sg0, sg1, si0, si1, so0, so1):
        idx_v = (idx_v0, idx_v1)
        rows_v = (rows_v0, rows_v1)
        obuf = (ob0, ob1)
        sg = (sg0, sg1)
        si = (si0, si1)
        so = (so0, so1)
        wid = lax.axis_index("s") * nc + lax.axis_index("c")
        u0 = wid * upw
        pltpu.sync_copy(pos_hbm, pos_v)

        def unit_su(n):
            u = u0 + n
            return u // upos, (u % upos) * uj

        def ids_src(n):
            s, j0 = unit_su(n)
            return ids_hbm.at[s, pl.ds(j0 * 128, ut)]

        def out_pairs(p, n):
            s, j0 = unit_su(n)
            return [
                (obuf[p].at[pl.ds(i * 8 * uj, 8 * uj), pl.ds(0, 128)],
                 out_hbm.at[s, pl.ds(i * bblk * 8 + j0 * 8, 8 * uj)])
                for i in range(dblk)
            ]

        def fire_out(p, n):
            for src, dst in out_pairs(p, n):
                pltpu.async_copy(src, dst, so[p])

        def wait_out(p, n):
            for src, dst in out_pairs(p, n):
                pltpu.make_async_copy(src, dst, so[p]).wait()

        def fire_gathers(p):
            for k in range(n_gather):
                pltpu.async_copy(
                    table_hbm.at[idx_v[p].at[pl.ds(k * gsz, gsz)]],
                    rows_v[p].at[pl.ds(k * gsz, gsz)],
                    sg[p],
                )

        def wait_gathers(p):
            for k in range(n_gather):
                pltpu.make_async_copy(
                    table_hbm.at[idx_v[p].at[pl.ds(k * gsz, gsz)]],
                    rows_v[p].at[pl.ds(k * gsz, gsz)],
                    sg[p],
                ).wait()

        iota = lax.iota(jnp.int32, LANES)
        row_pat = (iota // 8) * (8 * uj) + (iota % 8)

        def transpose_add(p, n):
            s, _ = unit_su(n)
            pv0 = pos_v[s, pl.ds(0, LANES)]
            pv1 = pos_v[s, pl.ds(LANES, LANES)]

            @plsc.parallel_loop(0, ut, unroll=16)
            def body(t):
                row0 = row_pat + jnp.full((LANES,), (t // 128) * 8, jnp.int32)
                row1 = row0 + 2 * (8 * uj)
                col = jnp.full((LANES,), t % 128, jnp.int32)
                v0 = rows_v[p][t, pl.ds(0, LANES)] + pv0
                v1 = rows_v[p][t, pl.ds(LANES, LANES)] + pv1
                plsc.store_scatter(obuf[p], [row0, col], v0)
                plsc.store_scatter(obuf[p], [row1, col], v1)

        pltpu.sync_copy(ids_src(0), idx_v[0])
        fire_gathers(0)
        pltpu.async_copy(ids_src(1), idx_v[1], si[1])

        def step(n, p):
            wait_gathers(p)

            @pl.when(n + 1 < upw)
            def _():
                pltpu.make_async_copy(ids_src(n + 1), idx_v[1 - p], si[1 - p]).wait()
                fire_gathers(1 - p)

            @pl.when(n + 2 < upw)
            def _():
                pltpu.async_copy(ids_src(n + 2), idx_v[p], si[p])

            @pl.when(n >= 2)
            def _():
                wait_out(p, n - 2)

            transpose_add(p, n)
            fire_out(p, n)

        def pair(m, carry):
            step(2 * m, 0)
            step(2 * m + 1, 1)
            return carry

        lax.fori_loop(0, upw // 2, pair, 0)
        wait_out(0, upw - 2)
        wait_out(1, upw - 1)

    return enc


def kernel(token_ids, token_table, pos_table):
    batch, seq = token_ids.shape
    vocab, dim = token_table.shape
    enc = _build_encoder(batch, seq, vocab, dim)
    ids_t = jnp.transpose(token_ids)
    out3 = enc(ids_t, token_table, pos_table)
    out5 = out3.reshape(seq, dim // 8, batch // 128, 8, 128)
    out = jnp.transpose(out5, (2, 4, 0, 1, 3)).reshape(batch, seq, dim)
    return out
